# Initial kernel scaffold; baseline (speedup 1.0000x reference)
import jax
import jax.numpy as jnp
from jax.experimental import pallas as pl


def kernel(agents_data, pos_pl, orient_pl, x_pl, agents_type, edge_index_pl2a, edge_index_a2a, params):
    raise NotImplementedError("write your pallas kernel here")


# speedup vs baseline: 1.8090x; 1.8090x over previous
import functools
import math

import jax
import jax.numpy as jnp
from jax import lax
from jax.experimental import pallas as pl
from jax.experimental.pallas import tpu as pltpu
from jax.experimental.pallas import tpu_sc as plsc

HID = 128
NFB = 64
NH = 8
HD = 16
NL = 2
_TR = 256
_NW = 32
_C = 40


def _lnb(x, g, b):
    mu = jnp.mean(x, axis=-1, keepdims=True)
    var = jnp.mean((x - mu) ** 2, axis=-1, keepdims=True)
    return (x - mu) / jnp.sqrt(var + 1e-5) * g + b


def _row2(v):
    return v.reshape(1, HID)


def _fourier_call(fp, cont, cat):
    R = cont.shape[0]
    ndim = fp["freqs"].shape[0]
    assert R % _TR == 0
    freqs = fp["freqs"]
    w1 = jnp.stack([m["l1"]["W"] for m in fp["mlps"]])
    b1 = jnp.stack([m["l1"]["b"] for m in fp["mlps"]])
    lg = jnp.stack([m["ln"]["g"] for m in fp["mlps"]])
    lb = jnp.stack([m["ln"]["b"] for m in fp["mlps"]])
    w2 = jnp.stack([m["l2"]["W"] for m in fp["mlps"]])
    b2 = jnp.stack([m["l2"]["b"] for m in fp["mlps"]])
    og = _row2(fp["out_ln"]["g"])
    ob = _row2(fp["out_ln"]["b"])
    wo = fp["out_lin"]["W"]
    bo = _row2(fp["out_lin"]["b"])
    has_cat = cat is not None

    def body(*refs):
        if has_cat:
            (c_ref, cat_ref, f_ref, w1_ref, b1_ref, lg_ref, lb_ref, w2_ref,
             b2_ref, og_ref, ob_ref, wo_ref, bo_ref, out_ref) = refs
        else:
            (c_ref, f_ref, w1_ref, b1_ref, lg_ref, lb_ref, w2_ref,
             b2_ref, og_ref, ob_ref, wo_ref, bo_ref, out_ref) = refs
        acc = jnp.zeros((_TR, HID), jnp.float32)
        for i in range(ndim):
            c = c_ref[:, i:i + 1]
            ang = c * (f_ref[i:i + 1, :] * (2.0 * math.pi))
            cs = jnp.concatenate([jnp.cos(ang), jnp.sin(ang)], axis=1)
            h = jnp.dot(cs, w1_ref[i, :2 * NFB, :],
                        preferred_element_type=jnp.float32)
            h = h + c * w1_ref[i, 2 * NFB:2 * NFB + 1, :]
            h = h + b1_ref[i:i + 1, :]
            h = jnp.maximum(_lnb(h, lg_ref[i:i + 1, :], lb_ref[i:i + 1, :]), 0.0)
            acc = acc + jnp.dot(h, w2_ref[i], preferred_element_type=jnp.float32)
            acc = acc + b2_ref[i:i + 1, :]
        if has_cat:
            acc = acc + cat_ref[...]
        acc = jnp.maximum(_lnb(acc, og_ref[...], ob_ref[...]), 0.0)
        out_ref[...] = jnp.dot(acc, wo_ref[...],
                               preferred_element_type=jnp.float32) + bo_ref[...]

    def full(a):
        return pl.BlockSpec(a.shape, lambda i: (0,) * a.ndim)

    ins = [cont] + ([cat] if has_cat else []) + [freqs, w1, b1, lg, lb, w2, b2,
                                                og, ob, wo, bo]
    in_specs = [pl.BlockSpec((_TR, 4), lambda i: (i, 0))]
    if has_cat:
        in_specs.append(pl.BlockSpec((_TR, HID), lambda i: (i, 0)))
    in_specs += [full(a) for a in [freqs, w1, b1, lg, lb, w2, b2, og, ob, wo, bo]]
    return pl.pallas_call(
        body,
        grid=(R // _TR,),
        in_specs=in_specs,
        out_specs=pl.BlockSpec((_TR, HID), lambda i: (i, 0)),
        out_shape=jax.ShapeDtypeStruct((R, HID), jnp.float32),
    )(*ins)


def _node_pre_call(x_src, x_dst, ap):
    R = x_src.shape[0]
    gs, bs = _row2(ap["pre_src"]["g"]), _row2(ap["pre_src"]["b"])
    gd, bd = _row2(ap["pre_dst"]["g"]), _row2(ap["pre_dst"]["b"])
    wq, bq = ap["to_q"]["W"], _row2(ap["to_q"]["b"])
    wk = ap["to_k"]["W"]
    wv, bv = ap["to_v"]["W"], _row2(ap["to_v"]["b"])
    ws, bs2 = ap["to_s"]["W"], _row2(ap["to_s"]["b"])

    def body(xs_ref, xd_ref, gs_r, bs_r, gd_r, bd_r, wq_r, bq_r, wk_r, wv_r,
             bv_r, ws_r, bs2_r, q_o, k_o, v_o, s_o):
        xs = _lnb(xs_ref[...], gs_r[...], bs_r[...])
        xd = _lnb(xd_ref[...], gd_r[...], bd_r[...])
        q_o[...] = jnp.dot(xd, wq_r[...], preferred_element_type=jnp.float32) + bq_r[...]
        k_o[...] = jnp.dot(xs, wk_r[...], preferred_element_type=jnp.float32)
        v_o[...] = jnp.dot(xs, wv_r[...], preferred_element_type=jnp.float32) + bv_r[...]
        s_o[...] = jnp.dot(xd, ws_r[...], preferred_element_type=jnp.float32) + bs2_r[...]

    def full(a):
        return pl.BlockSpec(a.shape, lambda i: (0,) * a.ndim)

    row = pl.BlockSpec((_TR, HID), lambda i: (i, 0))
    ws_list = [gs, bs, gd, bd, wq, bq, wk, wv, bv, ws, bs2]
    o_sh = jax.ShapeDtypeStruct((R, HID), jnp.float32)
    return pl.pallas_call(
        body,
        grid=(R // _TR,),
        in_specs=[row, row] + [full(a) for a in ws_list],
        out_specs=[row, row, row, row],
        out_shape=[o_sh, o_sh, o_sh, o_sh],
    )(x_src, x_dst, *ws_list)


def _edge_pre_call(r, aps):
    R = r.shape[0]
    gr = jnp.stack([_row2(ap["pre_r"]["g"]) for ap in aps])
    br = jnp.stack([_row2(ap["pre_r"]["b"]) for ap in aps])
    wkr = jnp.stack([ap["to_k_r"]["W"] for ap in aps])
    wvr = jnp.stack([ap["to_v_r"]["W"] for ap in aps])
    bvr = jnp.stack([_row2(ap["to_v_r"]["b"]) for ap in aps])

    def body(r_ref, gr_r, br_r, wkr_r, wvr_r, bvr_r, *outs):
        for l in range(NL):
            rr = _lnb(r_ref[...], gr_r[l], br_r[l])
            outs[2 * l][...] = jnp.dot(rr, wkr_r[l],
                                       preferred_element_type=jnp.float32)
            outs[2 * l + 1][...] = jnp.dot(rr, wvr_r[l],
                                           preferred_element_type=jnp.float32) + bvr_r[l]

    def full(a):
        return pl.BlockSpec(a.shape, lambda i: (0,) * a.ndim)

    row = pl.BlockSpec((_TR, HID), lambda i: (i, 0))
    o_sh = jax.ShapeDtypeStruct((R, HID), jnp.float32)
    outs = pl.pallas_call(
        body,
        grid=(R // _TR,),
        in_specs=[row] + [full(a) for a in [gr, br, wkr, wvr, bvr]],
        out_specs=[row] * (2 * NL),
        out_shape=[o_sh] * (2 * NL),
    )(r, gr, br, wkr, wvr, bvr)
    return outs


def _post_call(a0, a1, d0, d1, x_dst, s, ap):
    R = x_dst.shape[0]
    gd, bd = _row2(ap["pre_dst"]["g"]), _row2(ap["pre_dst"]["b"])
    wg = ap["to_g"]["W"]
    wg_a, wg_x = wg[:HID], wg[HID:]
    bg = _row2(ap["to_g"]["b"])
    wo, bo = ap["to_out"]["W"], _row2(ap["to_out"]["b"])
    gf, bf = _row2(ap["pre_ff"]["g"]), _row2(ap["pre_ff"]["b"])
    w1, b1 = ap["ff1"]["W"], ap["ff1"]["b"].reshape(1, 4 * HID)
    w2, b2 = ap["ff2"]["W"], _row2(ap["ff2"]["b"])

    def body(a0_r, a1_r, d0_r, d1_r, xd_r, s_r, gd_r, bd_r, wga_r, wgx_r,
             bg_r, wo_r, bo_r, gf_r, bf_r, w1_r, b1_r, w2_r, b2_r, out_r):
        agg = a0_r[...] + a1_r[...]
        den = d0_r[...] + d1_r[...]
        col = lax.broadcasted_iota(jnp.int32, (NH, HID), 1)
        rowi = lax.broadcasted_iota(jnp.int32, (NH, HID), 0)
        expand = (col // HD == rowi).astype(jnp.float32)
        denb = jnp.dot(den, expand, preferred_element_type=jnp.float32)
        aggn = agg / (denb + 1e-9)
        xd = _lnb(xd_r[...], gd_r[...], bd_r[...])
        g = jax.nn.sigmoid(
            jnp.dot(aggn, wga_r[...], preferred_element_type=jnp.float32)
            + jnp.dot(xd, wgx_r[...], preferred_element_type=jnp.float32)
            + bg_r[...])
        inputs = aggn + g * (s_r[...] - aggn)
        x = xd_r[...] + jnp.dot(inputs, wo_r[...],
                                preferred_element_type=jnp.float32) + bo_r[...]
        h = _lnb(x, gf_r[...], bf_r[...])
        h = jnp.maximum(jnp.dot(h, w1_r[...],
                                preferred_element_type=jnp.float32) + b1_r[...], 0.0)
        h = jnp.dot(h, w2_r[...], preferred_element_type=jnp.float32) + b2_r[...]
        out_r[...] = x + h

    def full(a):
        return pl.BlockSpec(a.shape, lambda i: (0,) * a.ndim)

    row = pl.BlockSpec((_TR, HID), lambda i: (i, 0))
    row16 = pl.BlockSpec((_TR, NH), lambda i: (i, 0))
    ws_list = [gd, bd, wg_a, wg_x, bg, wo, bo, gf, bf, w1, b1, w2, b2]
    return pl.pallas_call(
        body,
        grid=(R // _TR,),
        in_specs=[row, row, row16, row16, row, row] + [full(a) for a in ws_list],
        out_specs=row,
        out_shape=jax.ShapeDtypeStruct((R, HID), jnp.float32),
    )(a0, a1, d0, d1, x_dst, s, *ws_list)


def _sc_edge(src, dst, q, k, v, kr, vr, npad):
    E = src.shape[0]
    EW = E // _NW
    G = EW // _C
    rpt = npad // 16
    mesh = plsc.VectorSubcoreMesh(core_axis_name="c", subcore_axis_name="s")

    @functools.partial(
        pl.kernel,
        out_type=jax.ShapeDtypeStruct((2, npad, 136), jnp.float32),
        mesh=mesh,
        scratch_types=[
            pltpu.VMEM((_C,), jnp.int32),
            pltpu.VMEM((_C,), jnp.int32),
            pltpu.VMEM((_C, HID), jnp.float32),
            pltpu.VMEM((_C, HID), jnp.float32),
            pltpu.VMEM((_C, HID), jnp.float32),
            pltpu.VMEM((_C, HID), jnp.float32),
            pltpu.VMEM((_C, HID), jnp.float32),
            pltpu.VMEM((_C, 136), jnp.float32),
            pltpu.VMEM((64, 136), jnp.float32),
            pltpu.VMEM_SHARED((npad, 136), jnp.float32),
            pltpu.SemaphoreType.DMA,
        ],
        compiler_params=pltpu.CompilerParams(use_tc_tiling_on_sc=False),
    )
    def kfn(src_h, dst_h, q_h, k_h, v_h, kr_h, vr_h, out_h,
            idx_s, idx_d, qb, kb, vb, krb, vrb, mb, zb, shared, sem):
        c = lax.axis_index("c")
        s = lax.axis_index("s")
        wid = s * 2 + c

        @pl.loop(0, 64)
        def _zrow(i):
            for j in range(8):
                zb[i, pl.ds(j * 16, 16)] = jnp.zeros((16,), jnp.float32)
            zb[i, pl.ds(120, 16)] = jnp.zeros((16,), jnp.float32)

        @pl.loop(0, rpt // 64)
        def _zshared(t):
            pltpu.sync_copy(zb, shared.at[pl.ds(s * rpt + t * 64, 64)])

        plsc.subcore_barrier()

        @pl.loop(0, G)
        def _chunk(g):
            base = wid * EW + g * _C
            pltpu.sync_copy(src_h.at[pl.ds(base, _C)], idx_s)
            pltpu.sync_copy(dst_h.at[pl.ds(base, _C)], idx_d)
            cp1 = pltpu.async_copy(k_h.at[idx_s], kb, sem)
            cp2 = pltpu.async_copy(v_h.at[idx_s], vb, sem)
            cp3 = pltpu.async_copy(q_h.at[idx_d], qb, sem)
            pltpu.sync_copy(kr_h.at[pl.ds(base, _C)], krb)
            pltpu.sync_copy(vr_h.at[pl.ds(base, _C)], vrb)
            cp1.wait()
            cp2.wait()
            cp3.wait()

            @pl.loop(0, _C)
            def _edge(e):
                iota = lax.iota(jnp.int32, 16)
                rot8 = jnp.bitwise_xor(iota, 8)
                dv = jnp.zeros((16,), jnp.float32)
                mv7 = None
                for h in range(NH):
                    sl = pl.ds(h * HD, 16)
                    qv = qb[e, sl]
                    kv = kb[e, sl] + krb[e, sl]
                    p = qv * kv
                    for sh in (8, 4, 2, 1):
                        p = p + p[jnp.bitwise_xor(iota, sh)]
                    ev = jnp.exp(p * 0.25)
                    mv = ev * (vb[e, sl] + vrb[e, sl])
                    mb[e, sl] = mv
                    if h == NH - 1:
                        mv7 = mv
                    dv = dv + jnp.where(iota == h, ev, 0.0)
                mb[e, pl.ds(120, 16)] = jnp.where(iota < 8, mv7[rot8], dv[rot8])

            pltpu.sync_copy(mb, shared.at[idx_d], add=True)

        plsc.subcore_barrier()

        @pl.loop(0, rpt // 64)
        def _out(t):
            row = s * rpt + t * 64
            pltpu.sync_copy(shared.at[pl.ds(row, 64)], zb)
            pltpu.sync_copy(zb, out_h.at[c, pl.ds(row, 64)])

    return kfn(src, dst, q, k, v, kr, vr)


def _attn_layer(ap, x_src, x_dst, kr, vr, src, dst, npad):
    q, k, v, s = _node_pre_call(x_src, x_dst, ap)
    part = _sc_edge(src, dst, q, k, v, kr, vr, npad)
    a0 = part[0, :, :HID]
    a1 = part[1, :, :HID]
    d0 = part[0, :, HID:HID + NH]
    d1 = part[1, :, HID:HID + NH]
    return _post_call(a0, a1, d0, d1, x_dst, s, ap)


def _wrap_angle(a):
    return (a + math.pi) % (2.0 * math.pi) - math.pi


def _ang_between(ctr, nbr):
    return jnp.arctan2(ctr[:, 0] * nbr[:, 1] - ctr[:, 1] * nbr[:, 0],
                       jnp.sum(ctr * nbr, axis=-1))


def _pad_rows(x, r):
    return jnp.pad(x, ((0, r - x.shape[0]),) + ((0, 0),) * (x.ndim - 1))


def kernel(agents_data, pos_pl, orient_pl, x_pl, agents_type,
           edge_index_pl2a, edge_index_a2a, params):
    N = agents_data.shape[0]
    npad = -(-(N + 1) // 1024) * 1024

    pos_a = agents_data[:, 0:2]
    head_a = agents_data[:, 2]
    head_vec = jnp.stack([jnp.cos(head_a), jnp.sin(head_a)], axis=-1)
    vel = agents_data[:, 3:5]

    x_a_in = jnp.stack([jnp.linalg.norm(vel, axis=-1),
                        _ang_between(head_vec, vel)], axis=-1)
    cat = jnp.take(params["type_a_emb"], agents_type, axis=0)
    cont_a = _pad_rows(jnp.pad(x_a_in, ((0, 0), (0, 2))), npad)
    x_a = _fourier_call(params["x_a_emb"], cont_a, _pad_rows(cat, npad))

    s1 = edge_index_pl2a[0].astype(jnp.int32)
    d1 = edge_index_pl2a[1].astype(jnp.int32)
    rel_pos = pos_pl[s1] - pos_a[d1]
    rel_or = _wrap_angle(orient_pl[s1] - head_a[d1])
    r1_in = jnp.stack([jnp.linalg.norm(rel_pos, axis=-1),
                       _ang_between(head_vec[d1], rel_pos), rel_or], axis=-1)
    s2 = edge_index_a2a[0].astype(jnp.int32)
    d2 = edge_index_a2a[1].astype(jnp.int32)
    rp = pos_a[s2] - pos_a[d2]
    rh = _wrap_angle(head_a[s2] - head_a[d2])
    r2_in = jnp.stack([jnp.linalg.norm(rp, axis=-1),
                       _ang_between(head_vec[d2], rp), rh], axis=-1)

    E1 = s1.shape[0]
    E2 = s2.shape[0]
    e1p = -(-E1 // (_NW * _C)) * (_NW * _C)
    e2p = -(-E2 // (_NW * _C)) * (_NW * _C)
    er1 = -(-e1p // _TR) * _TR
    er2 = -(-e2p // _TR) * _TR

    r1 = _fourier_call(params["r_pl2a_emb"],
                       _pad_rows(jnp.pad(r1_in, ((0, 0), (0, 1))), er1), None)
    r2 = _fourier_call(params["r_a2a_emb"],
                       _pad_rows(jnp.pad(r2_in, ((0, 0), (0, 1))), er2), None)

    kr1_0, vr1_0, kr1_1, vr1_1 = _edge_pre_call(r1, params["pl2a"])
    kr2_0, vr2_0, kr2_1, vr2_1 = _edge_pre_call(r2, params["a2a"])
    kr1 = [kr1_0[:e1p], kr1_1[:e1p]]
    vr1 = [vr1_0[:e1p], vr1_1[:e1p]]
    kr2 = [kr2_0[:e2p], kr2_1[:e2p]]
    vr2 = [vr2_0[:e2p], vr2_1[:e2p]]

    def pad_idx(ix, ep, fill):
        return jnp.pad(ix, (0, ep - ix.shape[0]), constant_values=fill)

    s1p = pad_idx(s1, e1p, 0)
    d1p = pad_idx(d1, e1p, N)
    s2p = pad_idx(s2, e2p, 0)
    d2p = pad_idx(d2, e2p, N)

    x_pl_p = _pad_rows(x_pl, npad)
    for i in range(NL):
        x_a = _attn_layer(params["pl2a"][i], x_pl_p, x_a,
                          kr1[i], vr1[i], s1p, d1p, npad)
        x_a = _attn_layer(params["a2a"][i], x_a, x_a,
                          kr2[i], vr2[i], s2p, d2p, npad)
    return x_a[:N]
